# baseline (speedup 1.0000x reference)
import functools

import jax
import jax.numpy as jnp
from jax.experimental import pallas as pl
from jax.experimental.pallas import tpu as pltpu


def _glu_body(xa_ref, xb_ref, w_ref, b_ref, o_ref, *, rows, wo, stride, cin,
              cout, kh):
    wk = [w_ref[k] for k in range(kh)]
    bias = b_ref[...]
    kwin = w_ref.shape[2]
    step = stride * cin
    nrows = xa_ref.shape[0]

    for r in range(rows):
        for ow in range(wo):
            s = ow * step
            acc = jnp.zeros((2 * cout, o_ref.shape[3]), jnp.float32)
            for k in range(kh):
                lr = stride * r + k
                src = (xa_ref[lr, pl.ds(s, kwin), :] if lr < nrows
                       else xb_ref[0, pl.ds(s, kwin), :])
                acc = acc + jnp.dot(wk[k], src,
                                    preferred_element_type=jnp.float32)
            acc = acc + bias
            lin = acc[:cout, :]
            g = acc[cout:, :]
            z = jnp.exp(-jnp.abs(g))
            gate = jnp.where(g >= 0, 1.0, z) / (1.0 + z)
            o_ref[r, ow] = (lin * gate).astype(o_ref.dtype)


def _t_body(x_ref, o_ref):
    o_ref[...] = jnp.transpose(x_ref[...]).astype(o_ref.dtype)


def _transpose_2d(x2d, col_blk=2048):
    nn, mm = x2d.shape
    col_blk = min(col_blk, mm)
    grid = mm // col_blk
    return pl.pallas_call(
        _t_body,
        out_shape=jax.ShapeDtypeStruct((mm, nn), jnp.bfloat16),
        grid=(grid,),
        in_specs=[pl.BlockSpec((nn, col_blk), lambda i: (0, i))],
        out_specs=pl.BlockSpec((col_blk, nn), lambda i: (i, 0)),
        compiler_params=pltpu.CompilerParams(
            dimension_semantics=("parallel",),
            vmem_limit_bytes=64 * 1024 * 1024,
        ),
    )(x2d)


@functools.partial(jax.jit, static_argnames=("stride",))
def _glu_conv2d(x_nchw, w1, b1, w2, b2, *, stride):
    cout, cin, kh, kw = w1.shape
    n, _, h, w = x_nchw.shape
    ho = (h - kh) // stride + 1
    wo = (w - kw) // stride + 1

    nc = n // 2
    x_ra = jnp.transpose(x_nchw[:nc].astype(jnp.bfloat16),
                         (2, 3, 1, 0)).reshape(h, w * cin, nc)
    x_rb = jnp.transpose(x_nchw[nc:].astype(jnp.bfloat16),
                         (2, 3, 1, 0)).reshape(h, w * cin, nc)

    w_cat = jnp.concatenate([w1, w2], axis=0).astype(jnp.bfloat16)
    w_g = jnp.transpose(w_cat, (2, 0, 3, 1)).reshape(kh, 2 * cout, kw * cin)
    b_cat = jnp.concatenate([b1, b2]).reshape(2 * cout, 1)

    rows = 3
    nsteps = ho // rows
    body = functools.partial(_glu_body, rows=rows, wo=wo, stride=stride,
                             cin=cin, cout=cout, kh=kh)

    flops = 2 * n * ho * wo * 2 * cout * kh * kw * cin + 8 * n * cout * ho * wo
    cost = pl.CostEstimate(
        flops=flops,
        transcendentals=n * cout * ho * wo,
        bytes_accessed=4 * (n * cin * h * w + kh * 2 * cout * kw * cin
                            + 2 * cout + n * cout * ho * wo),
    )

    blk = stride * rows

    def conv_chunk(x_r):
        ncl = x_r.shape[2]
        return pl.pallas_call(
            body,
            out_shape=jax.ShapeDtypeStruct((ho, wo, cout, ncl), jnp.float32),
            grid=(nsteps,),
            in_specs=[
                pl.BlockSpec((blk, w * cin, ncl), lambda i: (i, 0, 0)),
                pl.BlockSpec((1, w * cin, ncl),
                             lambda i: (blk * i + blk, 0, 0)),
                pl.BlockSpec((kh, 2 * cout, kw * cin), lambda i: (0, 0, 0),
                             pipeline_mode=pl.Buffered(1)),
                pl.BlockSpec((2 * cout, 1), lambda i: (0, 0),
                             pipeline_mode=pl.Buffered(1)),
            ],
            out_specs=pl.BlockSpec((rows, wo, cout, ncl),
                                   lambda i: (i, 0, 0, 0)),
            compiler_params=pltpu.CompilerParams(
                dimension_semantics=("parallel",),
                vmem_limit_bytes=64 * 1024 * 1024,
            ),
            cost_estimate=cost,
        )(x_r, x_r, w_g, b_cat)

    out_a = conv_chunk(x_ra)
    out_b = conv_chunk(x_rb)

    return jnp.concatenate([jnp.transpose(out_a, (3, 2, 0, 1)),
                            jnp.transpose(out_b, (3, 2, 0, 1))], axis=0)


def kernel(x_nchw, w1, b1, w2, b2):
    return _glu_conv2d(x_nchw, w1, b1, w2, b2, stride=2)


# speedup vs baseline: 1.3206x; 1.3206x over previous
import functools

import jax
import jax.numpy as jnp
from jax.experimental import pallas as pl
from jax.experimental.pallas import tpu as pltpu


def _glu_body(xa_ref, xb_ref, w_ref, b_ref, o_ref, *, rows, wo, stride, cin,
              cout, kh):
    wk = [w_ref[k] for k in range(kh)]
    bias = b_ref[...]
    kwin = w_ref.shape[2]
    step = stride * cin
    nrows = xa_ref.shape[0]

    for r in range(rows):
        for ow in range(wo):
            s = ow * step
            acc = jnp.zeros((2 * cout, o_ref.shape[3]), jnp.float32)
            for k in range(kh):
                lr = stride * r + k
                src = (xa_ref[lr, pl.ds(s, kwin), :] if lr < nrows
                       else xb_ref[0, pl.ds(s, kwin), :])
                acc = acc + jnp.dot(wk[k], src,
                                    preferred_element_type=jnp.float32)
            acc = acc + bias
            lin = acc[:cout, :]
            g = acc[cout:, :]
            z = jnp.exp(-jnp.abs(g))
            gate = jnp.where(g >= 0, 1.0, z) / (1.0 + z)
            o_ref[r, ow] = (lin * gate).astype(o_ref.dtype)


def _t_body(x_ref, o_ref):
    o_ref[...] = jnp.transpose(x_ref[...]).astype(o_ref.dtype)


def _transpose_2d(x2d, col_blk=2048):
    nn, mm = x2d.shape
    col_blk = min(col_blk, mm)
    grid = mm // col_blk
    return pl.pallas_call(
        _t_body,
        out_shape=jax.ShapeDtypeStruct((mm, nn), jnp.bfloat16),
        grid=(grid,),
        in_specs=[pl.BlockSpec((nn, col_blk), lambda i: (0, i))],
        out_specs=pl.BlockSpec((col_blk, nn), lambda i: (i, 0)),
        compiler_params=pltpu.CompilerParams(
            dimension_semantics=("parallel",),
            vmem_limit_bytes=64 * 1024 * 1024,
        ),
    )(x2d)


@functools.partial(jax.jit, static_argnames=("stride",))
def _glu_conv2d(x_nchw, w1, b1, w2, b2, *, stride):
    cout, cin, kh, kw = w1.shape
    n, _, h, w = x_nchw.shape
    ho = (h - kh) // stride + 1
    wo = (w - kw) // stride + 1

    x_r = jnp.transpose(x_nchw.astype(jnp.bfloat16),
                        (2, 3, 1, 0)).reshape(h, w * cin, n)

    w_cat = jnp.concatenate([w1, w2], axis=0).astype(jnp.bfloat16)
    w_g = jnp.transpose(w_cat, (2, 0, 3, 1)).reshape(kh, 2 * cout, kw * cin)
    b_cat = jnp.concatenate([b1, b2]).reshape(2 * cout, 1)

    rows = 3
    nsteps = ho // rows
    body = functools.partial(_glu_body, rows=rows, wo=wo, stride=stride,
                             cin=cin, cout=cout, kh=kh)

    flops = 2 * n * ho * wo * 2 * cout * kh * kw * cin + 8 * n * cout * ho * wo
    cost = pl.CostEstimate(
        flops=flops,
        transcendentals=n * cout * ho * wo,
        bytes_accessed=4 * (n * cin * h * w + kh * 2 * cout * kw * cin
                            + 2 * cout + n * cout * ho * wo),
    )

    blk = stride * rows
    out = pl.pallas_call(
        body,
        out_shape=jax.ShapeDtypeStruct((ho, wo, cout, n), jnp.float32),
        grid=(nsteps,),
        in_specs=[
            pl.BlockSpec((blk, w * cin, n), lambda i: (i, 0, 0)),
            pl.BlockSpec((1, w * cin, n), lambda i: (blk * i + blk, 0, 0)),
            pl.BlockSpec((kh, 2 * cout, kw * cin), lambda i: (0, 0, 0),
                         pipeline_mode=pl.Buffered(1)),
            pl.BlockSpec((2 * cout, 1), lambda i: (0, 0),
                         pipeline_mode=pl.Buffered(1)),
        ],
        out_specs=pl.BlockSpec((rows, wo, cout, n), lambda i: (i, 0, 0, 0)),
        compiler_params=pltpu.CompilerParams(
            dimension_semantics=("parallel",),
            vmem_limit_bytes=64 * 1024 * 1024,
        ),
        cost_estimate=cost,
    )(x_r, x_r, w_g, b_cat)

    return jnp.transpose(out, (3, 2, 0, 1))


def kernel(x_nchw, w1, b1, w2, b2):
    return _glu_conv2d(x_nchw, w1, b1, w2, b2, stride=2)
